# baseline (speedup 1.0000x reference)
import jax
import jax.numpy as jnp
from jax.experimental import pallas as pl
from jax.experimental.pallas import tpu as pltpu

NUM_PROMPTS = 50
PROMPT_DIM = 768
SEQ = 577
_BB = 4


def _splice_kernel(layer_ref, x_ref, pe_ref, o_ref):
    del layer_ref
    o_ref[:, 0:1, :] = x_ref[:, 0:1, :]
    o_ref[:, 1:1 + NUM_PROMPTS, :] = jnp.broadcast_to(
        pe_ref[...], (_BB, NUM_PROMPTS, PROMPT_DIM))
    o_ref[:, 1 + NUM_PROMPTS:, :] = x_ref[:, 1:, :]


def kernel(x, prompt_embeddings, layer_num):
    Bsz = x.shape[0]
    layer = jnp.asarray(layer_num, jnp.int32).reshape((1,))
    grid_spec = pltpu.PrefetchScalarGridSpec(
        num_scalar_prefetch=1,
        grid=(Bsz // _BB,),
        in_specs=[
            pl.BlockSpec((_BB, SEQ, PROMPT_DIM), lambda b, s: (b, 0, 0)),
            pl.BlockSpec((1, NUM_PROMPTS, PROMPT_DIM),
                         lambda b, s: (s[0], 0, 0)),
        ],
        out_specs=pl.BlockSpec((_BB, SEQ + NUM_PROMPTS, PROMPT_DIM),
                               lambda b, s: (b, 0, 0)),
    )
    return pl.pallas_call(
        _splice_kernel,
        grid_spec=grid_spec,
        out_shape=jax.ShapeDtypeStruct((Bsz, SEQ + NUM_PROMPTS, PROMPT_DIM),
                                       x.dtype),
        compiler_params=pltpu.CompilerParams(
            dimension_semantics=("parallel",)),
    )(layer, x, prompt_embeddings)
